# baseline (speedup 1.0000x reference)
import functools
import math

import jax
import jax.numpy as jnp
from jax import lax
from jax.experimental import pallas as pl
from jax.experimental.pallas import tpu as pltpu
from jax.experimental.pallas import tpu_sc as plsc

D_MODEL = 64
SCALE = math.sqrt(D_MODEL)
NUM_CORES = 2
NUM_SUBCORES = 16
NUM_WORKERS = NUM_CORES * NUM_SUBCORES
LANES = 16
N_I = 4096
N_J = 200
B_TOTAL = N_I * N_J
PER_WORKER = B_TOTAL // NUM_WORKERS
CHUNK = 256
NCHUNKS = PER_WORKER // CHUNK
TILES_PER_CHUNK = CHUNK // 128
VECS = CHUNK * D_MODEL // LANES


def _embed(idx_flat, table):
    mesh = plsc.VectorSubcoreMesh(
        core_axis_name="c", subcore_axis_name="s",
        num_cores=NUM_CORES, num_subcores=NUM_SUBCORES)

    @functools.partial(
        pl.kernel,
        out_type=jax.ShapeDtypeStruct(
            (N_J, D_MODEL // 8, N_I // 128, 8, 128), jnp.float32),
        mesh=mesh,
        scratch_types=[
            pltpu.VMEM((PER_WORKER,), jnp.int32),
            pltpu.VMEM((CHUNK, D_MODEL), jnp.float32),
            pltpu.VMEM((CHUNK, D_MODEL), jnp.float32),
            pltpu.VMEM((D_MODEL // 8, TILES_PER_CHUNK, 8, 128), jnp.float32),
            pltpu.VMEM((D_MODEL // 8, TILES_PER_CHUNK, 8, 128), jnp.float32),
            pltpu.SemaphoreType.DMA,
            pltpu.SemaphoreType.DMA,
        ],
        compiler_params=pltpu.CompilerParams(
            use_tc_tiling_on_sc=False, needs_layout_passes=False),
    )
    def emb_kernel(idx_hbm, table_hbm, out_hbm, idx_v, rows0, rows1,
                   t0, t1, s0, s1):
        wid = lax.axis_index("s") * NUM_CORES + lax.axis_index("c")
        base = wid * PER_WORKER
        pltpu.sync_copy(idx_hbm.at[pl.ds(base, PER_WORKER)], idx_v)

        bufs = (rows0, rows1)
        tbufs = (t0, t1)
        sems = (s0, s1)
        iota = lax.iota(jnp.int32, LANES)

        def gather_start(g, b):
            pltpu.async_copy(
                table_hbm.at[idx_v.at[pl.ds(g * CHUNK, CHUNK)]],
                bufs[b], sems[b])

        def transpose_store(g, b):
            rows = bufs[b]
            tb = tbufs[b]

            def tr_body(v):
                c = v // (CHUNK // LANES)
                k = v % (CHUNK // LANES)
                row_idx = k * LANES + iota
                col_idx = jnp.full((LANES,), 0, jnp.int32) + c
                vals = plsc.load_gather(rows, [row_idx, col_idx])
                tb[c // 8, k // 8, c % 8, pl.ds((k % 8) * LANES, LANES)] = (
                    vals * SCALE)

            plsc.parallel_loop(0, VECS, 1, unroll=8)(tr_body)

            m0 = base + g * CHUNK
            j = m0 // N_I
            it0 = (m0 % N_I) // 128
            pltpu.sync_copy(tb, out_hbm.at[j, :, pl.ds(it0, TILES_PER_CHUNK)])

        gather_start(0, 0)

        def pair_body(h, carry):
            for b in range(2):
                g = 2 * h + b

                @pl.when(g + 1 < NCHUNKS)
                def _start_next():
                    gather_start(g + 1, 1 - b)

                pltpu.make_async_copy(
                    table_hbm.at[idx_v.at[pl.ds(g * CHUNK, CHUNK)]],
                    bufs[b], sems[b]).wait()
                transpose_store(g, b)
            return carry

        lax.fori_loop(0, NCHUNKS // 2, pair_body, 0)

    return emb_kernel(idx_flat, table)


def kernel(x, table):
    idx_flat = x.T.reshape(-1).astype(jnp.int32)
    out5 = _embed(idx_flat, table)
    return (out5.transpose(2, 4, 0, 1, 3)
            .reshape(N_I, N_J, D_MODEL))


# speedup vs baseline: 1.0732x; 1.0732x over previous
import functools
import math

import jax
import jax.numpy as jnp
from jax import lax
from jax.experimental import pallas as pl
from jax.experimental.pallas import tpu as pltpu
from jax.experimental.pallas import tpu_sc as plsc

D_MODEL = 64
SCALE = math.sqrt(D_MODEL)
NUM_CORES = 2
NUM_SUBCORES = 16
NUM_WORKERS = NUM_CORES * NUM_SUBCORES
LANES = 16
N_I = 4096
N_J = 200
B_TOTAL = N_I * N_J
PER_WORKER = B_TOTAL // NUM_WORKERS
CHUNK = 512
NCHUNKS = PER_WORKER // CHUNK
TILES_PER_CHUNK = CHUNK // 128


def _embed(idx_flat, table):
    mesh = plsc.VectorSubcoreMesh(
        core_axis_name="c", subcore_axis_name="s",
        num_cores=NUM_CORES, num_subcores=NUM_SUBCORES)

    @functools.partial(
        pl.kernel,
        out_type=jax.ShapeDtypeStruct(
            (N_J, D_MODEL // 8, N_I // 128, 8, 128), jnp.float32),
        mesh=mesh,
        scratch_types=[
            pltpu.VMEM((CHUNK,), jnp.int32),
            pltpu.VMEM((CHUNK,), jnp.int32),
            pltpu.VMEM((CHUNK, D_MODEL), jnp.float32),
            pltpu.VMEM((CHUNK, D_MODEL), jnp.float32),
            pltpu.VMEM((D_MODEL // 8, TILES_PER_CHUNK, 8, 128), jnp.float32),
            pltpu.SemaphoreType.DMA,
            pltpu.SemaphoreType.DMA,
        ],
        compiler_params=pltpu.CompilerParams(
            use_tc_tiling_on_sc=False, needs_layout_passes=False),
    )
    def emb_kernel(idx_hbm, table_hbm, out_hbm, i0, i1, rows0, rows1,
                   tb, s0, s1):
        wid = lax.axis_index("s") * NUM_CORES + lax.axis_index("c")
        base = wid * PER_WORKER

        idxb = (i0, i1)
        bufs = (rows0, rows1)
        sems = (s0, s1)
        iota = lax.iota(jnp.int32, LANES)

        def gather_start(g, b):
            pltpu.sync_copy(idx_hbm.at[pl.ds(base + g * CHUNK, CHUNK)],
                            idxb[b])
            pltpu.async_copy(table_hbm.at[idxb[b]], bufs[b], sems[b])

        def transpose_store(g, b):
            rows = bufs[b]

            def tr_body(k):
                row_idx = k * LANES + iota
                t = k // 8
                kk = (k % 8) * LANES
                for c in range(D_MODEL):
                    col = jnp.full((LANES,), c, jnp.int32)
                    vals = plsc.load_gather(rows, [row_idx, col])
                    tb[c // 8, t, c % 8, pl.ds(kk, LANES)] = vals * SCALE

            plsc.parallel_loop(0, CHUNK // LANES, 1, unroll=2)(tr_body)

            m0 = base + g * CHUNK
            j = m0 // N_I
            it0 = (m0 % N_I) // 128
            pltpu.sync_copy(tb, out_hbm.at[j, :, pl.ds(it0, TILES_PER_CHUNK)])

        gather_start(0, 0)

        def pair_body(h, carry):
            for b in range(2):
                g = 2 * h + b

                @pl.when(g + 1 < NCHUNKS)
                def _start_next():
                    gather_start(g + 1, 1 - b)

                pltpu.make_async_copy(
                    table_hbm.at[idxb[b]], bufs[b], sems[b]).wait()
                transpose_store(g, b)
            return carry

        lax.fori_loop(0, NCHUNKS // 2, pair_body, 0)

    return emb_kernel(idx_flat, table)


def kernel(x, table):
    idx_flat = x.T.reshape(-1).astype(jnp.int32)
    out5 = _embed(idx_flat, table)
    return (out5.transpose(2, 4, 0, 1, 3)
            .reshape(N_I, N_J, D_MODEL))


# speedup vs baseline: 1.9595x; 1.8259x over previous
import functools
import math

import jax
import jax.numpy as jnp
from jax import lax
from jax.experimental import pallas as pl
from jax.experimental.pallas import tpu as pltpu
from jax.experimental.pallas import tpu_sc as plsc

D_MODEL = 64
SCALE = math.sqrt(D_MODEL)
NUM_CORES = 2
NUM_SUBCORES = 16
NUM_WORKERS = NUM_CORES * NUM_SUBCORES
LANES = 16
N_I = 4096
N_J = 200
B_TOTAL = N_I * N_J
PER_WORKER = B_TOTAL // NUM_WORKERS
CHUNK = 256
NCHUNKS = PER_WORKER // CHUNK
TILES_PER_CHUNK = CHUNK // 128
PITCH = 129


def _embed(idx_flat, table):
    mesh = plsc.VectorSubcoreMesh(
        core_axis_name="c", subcore_axis_name="s",
        num_cores=NUM_CORES, num_subcores=NUM_SUBCORES)

    @functools.partial(
        pl.kernel,
        out_type=jax.ShapeDtypeStruct(
            (N_J, D_MODEL // 8, N_I // 128, 8, 128), jnp.float32),
        mesh=mesh,
        scratch_types=[
            pltpu.VMEM((CHUNK,), jnp.int32),
            pltpu.VMEM((CHUNK,), jnp.int32),
            pltpu.VMEM((CHUNK, D_MODEL), jnp.float32),
            pltpu.VMEM((CHUNK, D_MODEL), jnp.float32),
            pltpu.VMEM((D_MODEL // 8, TILES_PER_CHUNK, 8, PITCH), jnp.float32),
            pltpu.VMEM((D_MODEL // 8, TILES_PER_CHUNK, 8, PITCH), jnp.float32),
            pltpu.SemaphoreType.DMA,
            pltpu.SemaphoreType.DMA,
        ],
        compiler_params=pltpu.CompilerParams(
            use_tc_tiling_on_sc=False, needs_layout_passes=False),
    )
    def emb_kernel(idx_hbm, table_hbm, out_hbm, i0, i1, rows0, rows1,
                   tb0, tb1, s0, s1):
        wid = lax.axis_index("s") * NUM_CORES + lax.axis_index("c")
        base = wid * PER_WORKER

        idxb = (i0, i1)
        bufs = (rows0, rows1)
        tbufs = (tb0, tb1)
        sems = (s0, s1)
        iota = lax.iota(jnp.int32, LANES)
        blk_base = iota // 8
        cin_vec = iota % 8

        def gather_start(g, b):
            pltpu.sync_copy(idx_hbm.at[pl.ds(base + g * CHUNK, CHUNK)],
                            idxb[b])
            pltpu.async_copy(table_hbm.at[idxb[b]], bufs[b], sems[b])

        def transpose_store(g, b):
            rows = bufs[b]
            tb = tbufs[b]

            def tr_body(i):
                t = i // 128
                t_vec = jnp.full((LANES,), 0, jnp.int32) + t
                iin_vec = jnp.full((LANES,), 0, jnp.int32) + (i % 128)
                for g8 in range(D_MODEL // LANES):
                    vals = rows[i, pl.ds(g8 * LANES, LANES)]
                    plsc.store_scatter(
                        tb,
                        [blk_base + g8 * 2, t_vec, cin_vec, iin_vec],
                        vals * SCALE)

            plsc.parallel_loop(0, CHUNK, 1, unroll=4)(tr_body)

            m0 = base + g * CHUNK
            j = m0 // N_I
            it0 = (m0 % N_I) // 128
            pltpu.sync_copy(tb.at[:, :, :, pl.ds(0, 128)],
                            out_hbm.at[j, :, pl.ds(it0, TILES_PER_CHUNK)])

        gather_start(0, 0)

        def pair_body(h, carry):
            for b in range(2):
                g = 2 * h + b

                @pl.when(g + 1 < NCHUNKS)
                def _start_next():
                    gather_start(g + 1, 1 - b)

                pltpu.make_async_copy(
                    table_hbm.at[idxb[b]], bufs[b], sems[b]).wait()
                transpose_store(g, b)
            return carry

        lax.fori_loop(0, NCHUNKS // 2, pair_body, 0)

    return emb_kernel(idx_flat, table)


def kernel(x, table):
    idx_flat = x.T.reshape(-1).astype(jnp.int32)
    out5 = _embed(idx_flat, table)
    return (out5.transpose(2, 4, 0, 1, 3)
            .reshape(N_I, N_J, D_MODEL))


# speedup vs baseline: 1.9596x; 1.0000x over previous
import functools
import math

import jax
import jax.numpy as jnp
from jax import lax
from jax.experimental import pallas as pl
from jax.experimental.pallas import tpu as pltpu
from jax.experimental.pallas import tpu_sc as plsc

D_MODEL = 64
SCALE = math.sqrt(D_MODEL)
NUM_CORES = 2
NUM_SUBCORES = 16
NUM_WORKERS = NUM_CORES * NUM_SUBCORES
LANES = 16
N_I = 4096
N_J = 200
B_TOTAL = N_I * N_J
PER_WORKER = B_TOTAL // NUM_WORKERS
CHUNK = 256
NCHUNKS = PER_WORKER // CHUNK
TILES_PER_CHUNK = CHUNK // 128
PITCH = 129


def _embed(idx_flat, table):
    mesh = plsc.VectorSubcoreMesh(
        core_axis_name="c", subcore_axis_name="s",
        num_cores=NUM_CORES, num_subcores=NUM_SUBCORES)

    @functools.partial(
        pl.kernel,
        out_type=jax.ShapeDtypeStruct(
            (N_J, D_MODEL // 8, N_I // 128, 8, 128), jnp.float32),
        mesh=mesh,
        scratch_types=[
            pltpu.VMEM((CHUNK,), jnp.int32),
            pltpu.VMEM((CHUNK,), jnp.int32),
            pltpu.VMEM((CHUNK, D_MODEL), jnp.float32),
            pltpu.VMEM((CHUNK, D_MODEL), jnp.float32),
            pltpu.VMEM((D_MODEL // 8, TILES_PER_CHUNK, 8, PITCH), jnp.float32),
            pltpu.VMEM((D_MODEL // 8, TILES_PER_CHUNK, 8, PITCH), jnp.float32),
            pltpu.SemaphoreType.DMA,
            pltpu.SemaphoreType.DMA,
        ],
        compiler_params=pltpu.CompilerParams(
            use_tc_tiling_on_sc=False, needs_layout_passes=False),
    )
    def emb_kernel(idx_hbm, table_hbm, out_hbm, i0, i1, rows0, rows1,
                   tb0, tb1, s0, s1):
        wid = lax.axis_index("s") * NUM_CORES + lax.axis_index("c")
        base = wid * PER_WORKER

        idxb = (i0, i1)
        bufs = (rows0, rows1)
        tbufs = (tb0, tb1)
        sems = (s0, s1)
        iota = lax.iota(jnp.int32, LANES)
        blk_base = iota // 8
        cin_vec = iota % 8

        def gather_start(g, b):
            pltpu.sync_copy(idx_hbm.at[pl.ds(base + g * CHUNK, CHUNK)],
                            idxb[b])
            pltpu.async_copy(table_hbm.at[idxb[b]], bufs[b], sems[b])

        def transpose_store(g, b):
            rows = bufs[b]
            tb = tbufs[b]

            def tr_body(i):
                t = i // 128
                t_vec = jnp.full((LANES,), 0, jnp.int32) + t
                iin_vec = jnp.full((LANES,), 0, jnp.int32) + (i % 128)
                for g8 in range(D_MODEL // LANES):
                    vals = rows[i, pl.ds(g8 * LANES, LANES)]
                    plsc.store_scatter(
                        tb,
                        [blk_base + g8 * 2, t_vec, cin_vec, iin_vec],
                        vals * SCALE)

            plsc.parallel_loop(0, CHUNK, 1, unroll=8)(tr_body)

            m0 = base + g * CHUNK
            j = m0 // N_I
            it0 = (m0 % N_I) // 128
            pltpu.sync_copy(tb.at[:, :, :, pl.ds(0, 128)],
                            out_hbm.at[j, :, pl.ds(it0, TILES_PER_CHUNK)])

        gather_start(0, 0)

        def pair_body(h, carry):
            for b in range(2):
                g = 2 * h + b

                @pl.when(g + 1 < NCHUNKS)
                def _start_next():
                    gather_start(g + 1, 1 - b)

                pltpu.make_async_copy(
                    table_hbm.at[idxb[b]], bufs[b], sems[b]).wait()
                transpose_store(g, b)
            return carry

        lax.fori_loop(0, NCHUNKS // 2, pair_body, 0)

    return emb_kernel(idx_flat, table)


def kernel(x, table):
    idx_flat = x.T.reshape(-1).astype(jnp.int32)
    out5 = _embed(idx_flat, table)
    return (out5.transpose(2, 4, 0, 1, 3)
            .reshape(N_I, N_J, D_MODEL))
